# baseline (speedup 1.0000x reference)
import jax
import jax.numpy as jnp
from jax import lax
from jax.experimental import pallas as pl
from jax.experimental.pallas import tpu as pltpu
from jax.experimental.pallas import tpu_sc as plsc

B = 32
P = 16
PS = 32
D = 64
SR = 2048
SL = 1024
CH = 2 * D + 1
L = 16

NC = 2
NS = 16
NW = NC * NS
TASKS = B * P
TPW = TASKS // NW

AC = 4
NBUF = 2
GROUPS = PS // (AC * NBUF)
SPAIRS = 32


def _sc_body(seq1_hbm, seq2_hbm, desc_hbm, geo_hbm, out_hbm,
             descw_v, row_v0, col_v0, geo_v0, row_v1, col_v1, geo_v1,
             buf0, buf1,
             sem_in0, sem_in1, sem0, sem1):
    c = lax.axis_index("c")
    s = lax.axis_index("s")
    wid = s * NC + c
    base_task = wid * TPW
    bufs = (buf0, buf1)
    sems = (sem0, sem1)
    row_vs = (row_v0, row_v1)
    col_vs = (col_v0, col_v1)
    geo_vs = (geo_v0, geo_v1)
    sem_ins = (sem_in0, sem_in1)
    iota = lax.iota(jnp.int32, L)

    pltpu.sync_copy(desc_hbm.at[pl.ds(pl.multiple_of(base_task, 8), TPW)],
                    descw_v)

    def prefetch(tl, par):
        task = base_task + tl
        i = task // P
        p = lax.rem(task, P)
        pvec = descw_v[tl, pl.ds(0, L)]
        q8r = pl.multiple_of(pvec[0], 8)
        q8c = pl.multiple_of(pvec[2], 8)
        pltpu.async_copy(seq1_hbm.at[i, pl.ds(q8r, SPAIRS), :],
                         row_vs[par], sem_ins[par])
        pltpu.async_copy(seq2_hbm.at[i, pl.ds(q8c, SPAIRS), :],
                         col_vs[par], sem_ins[par])
        pltpu.async_copy(geo_hbm.at[i, p], geo_vs[par], sem_ins[par])

    def wait_inputs(par):
        pltpu.make_async_copy(
            seq1_hbm.at[0, pl.ds(0, SPAIRS), :], row_vs[par],
            sem_ins[par]).wait()
        pltpu.make_async_copy(
            seq2_hbm.at[0, pl.ds(0, SPAIRS), :], col_vs[par],
            sem_ins[par]).wait()
        pltpu.make_async_copy(
            geo_hbm.at[0, 0], geo_vs[par], sem_ins[par]).wait()

    prefetch(0, 0)
    prefetch(1, 1)

    def process(u, tl, par):
        task = base_task + tl
        i = task // P
        p = lax.rem(task, P)
        pvec = descw_v[tl, pl.ds(0, L)]
        ro = pvec[1]
        co = pvec[3]
        wait_inputs(par)
        row_v = row_vs[par]
        col_v = col_vs[par]
        geo_v = geo_vs[par]

        def fill_group(g, _):
            for k in range(NBUF):
                buf = bufs[k]
                a0 = (g * NBUF + k) * AC

                if par == 0:
                    not_first = jnp.logical_not((u == 0) & (g == 0))
                else:
                    not_first = jnp.bool_(True)

                @pl.when(not_first)
                def _reclaim(buf=buf, sem=sems[k]):
                    pltpu.make_async_copy(
                        out_hbm.at[0, 0, pl.ds(0, AC)], buf, sem).wait()
                for b in range(PS):
                    pr = (ro + b) // 2
                    hf = ((ro + b) % 2) * D
                    for j in range(D // L):
                        xr = row_v[pr, pl.ds(hf + j * L, L)]
                        for m in range(AC):
                            buf[m, b, pl.ds(j * L, L)] = xr
                for m in range(AC):
                    a = a0 + m
                    pc = (co + a) // 2
                    hc = ((co + a) % 2) * D
                    for j in range(D // L):
                        xc = col_v[pc, pl.ds(hc + j * L, L)]
                        for b in range(PS):
                            buf[m, b, pl.ds(D + j * L, L)] = xc
                    gs = a // 4
                    go = (a % 4) * PS
                    for h in range(PS // L):
                        xg = geo_v[gs, pl.ds(go + h * L, L)]
                        plsc.store_scatter(
                            buf,
                            [jnp.full((L,), m, jnp.int32),
                             iota + (h * L),
                             jnp.full((L,), CH - 1, jnp.int32)],
                            xg)
                pltpu.async_copy(buf, out_hbm.at[i, p, pl.ds(a0, AC)], sems[k])
            return ()

        lax.fori_loop(0, GROUPS, fill_group, (), unroll=False)
        prefetch(jnp.minimum(tl + 2, TPW - 1), par)

    def pair_body(u, _):
        process(u, u * 2, 0)
        process(u, u * 2 + 1, 1)
        return ()

    lax.fori_loop(0, TPW // 2, pair_body, (), unroll=False)

    for k in range(NBUF):
        pltpu.make_async_copy(
            out_hbm.at[0, 0, pl.ds(0, AC)], bufs[k], sems[k]).wait()
    wait_inputs(0)
    wait_inputs(1)


def kernel(seq1M, seq2M, patches, geo):
    seq1p = seq1M.reshape(B, SR // 2, 2 * D)
    seq2p = seq2M.reshape(B, SL // 2, 2 * D)
    geo8 = geo.reshape(B, P, PS * PS // 128, 128)
    r = patches[:, :, 0].reshape(TASKS).astype(jnp.int32)
    cc = patches[:, :, 1].reshape(TASKS).astype(jnp.int32)
    q8r = jnp.minimum((r // 16) * 8, SR // 2 - SPAIRS)
    q8c = jnp.minimum((cc // 16) * 8, SL // 2 - SPAIRS)
    desc = jnp.stack([q8r, r - 2 * q8r, q8c, cc - 2 * q8c], axis=1)
    desc = jnp.pad(desc, ((0, 0), (0, 128 - 4)))

    run = pl.kernel(
        _sc_body,
        out_type=jax.ShapeDtypeStruct((B, P, PS, PS, CH), jnp.float32),
        mesh=plsc.VectorSubcoreMesh(core_axis_name="c", subcore_axis_name="s"),
        compiler_params=pltpu.CompilerParams(needs_layout_passes=False),
        scratch_types=[
            pltpu.VMEM((TPW, 128), jnp.int32),
            pltpu.VMEM((SPAIRS, 2 * D), jnp.float32),
            pltpu.VMEM((SPAIRS, 2 * D), jnp.float32),
            pltpu.VMEM((PS * PS // 128, 128), jnp.float32),
            pltpu.VMEM((SPAIRS, 2 * D), jnp.float32),
            pltpu.VMEM((SPAIRS, 2 * D), jnp.float32),
            pltpu.VMEM((PS * PS // 128, 128), jnp.float32),
            pltpu.VMEM((AC, PS, CH), jnp.float32),
            pltpu.VMEM((AC, PS, CH), jnp.float32),
            pltpu.SemaphoreType.DMA,
            pltpu.SemaphoreType.DMA,
            pltpu.SemaphoreType.DMA,
            pltpu.SemaphoreType.DMA,
        ],
    )
    return run(seq1p, seq2p, desc, geo8)


# speedup vs baseline: 1.0815x; 1.0815x over previous
import jax
import jax.numpy as jnp
from jax import lax
from jax.experimental import pallas as pl
from jax.experimental.pallas import tpu as pltpu
from jax.experimental.pallas import tpu_sc as plsc

B = 32
P = 16
PS = 32
D = 64
SR = 2048
SL = 1024
CH = 2 * D + 1
L = 16

NC = 2
NS = 16
NW = NC * NS
TASKS = B * P
TPW = TASKS // NW

AC = 4
NBUF = 2
GROUPS = PS // (AC * NBUF)
SPAIRS = 32


def _sc_body(seq1_hbm, seq2_hbm, desc_hbm, geo_hbm, out_hbm,
             pv, row_v, col_v, geo_v,
             buf0, buf1,
             sem_in, sem0, sem1):
    c = lax.axis_index("c")
    s = lax.axis_index("s")
    wid = s * NC + c
    bufs = (buf0, buf1)
    sems = (sem0, sem1)
    iota = lax.iota(jnp.int32, L)

    def task_body(t, _):
        task = wid * TPW + t
        i = task // P
        p = lax.rem(task, P)
        pltpu.sync_copy(desc_hbm.at[task], pv)
        pvec = pv[pl.ds(0, L)]
        q8r = pl.multiple_of(pvec[0], 8)
        ro = pvec[1]
        q8c = pl.multiple_of(pvec[2], 8)
        co = pvec[3]
        pltpu.async_copy(seq1_hbm.at[i, pl.ds(q8r, SPAIRS), :], row_v, sem_in)
        pltpu.async_copy(seq2_hbm.at[i, pl.ds(q8c, SPAIRS), :], col_v, sem_in)
        pltpu.sync_copy(geo_hbm.at[i, p], geo_v)
        pltpu.make_async_copy(
            seq1_hbm.at[i, pl.ds(0, SPAIRS), :], row_v, sem_in).wait()
        pltpu.make_async_copy(
            seq2_hbm.at[i, pl.ds(0, SPAIRS), :], col_v, sem_in).wait()

        def fill_group(g, _):
            for k in range(NBUF):
                buf = bufs[k]
                a0 = (g * NBUF + k) * AC

                @pl.when(jnp.logical_not((t == 0) & (g == 0)))
                def _reclaim(buf=buf, sem=sems[k]):
                    pltpu.make_async_copy(
                        out_hbm.at[0, 0, pl.ds(0, AC)], buf, sem).wait()
                for b in range(PS):
                    pr = (ro + b) // 2
                    hf = ((ro + b) % 2) * D
                    for j in range(D // L):
                        xr = row_v[pr, pl.ds(hf + j * L, L)]
                        for m in range(AC):
                            buf[m, b, pl.ds(j * L, L)] = xr
                for m in range(AC):
                    a = a0 + m
                    pc = (co + a) // 2
                    hc = ((co + a) % 2) * D
                    for j in range(D // L):
                        xc = col_v[pc, pl.ds(hc + j * L, L)]
                        for b in range(PS):
                            buf[m, b, pl.ds(D + j * L, L)] = xc
                    gs = a // 4
                    go = (a % 4) * PS
                    for h in range(PS // L):
                        xg = geo_v[gs, pl.ds(go + h * L, L)]
                        plsc.store_scatter(
                            buf,
                            [jnp.full((L,), m, jnp.int32),
                             iota + (h * L),
                             jnp.full((L,), CH - 1, jnp.int32)],
                            xg)
                pltpu.async_copy(buf, out_hbm.at[i, p, pl.ds(a0, AC)], sems[k])
            return ()

        lax.fori_loop(0, GROUPS, fill_group, (), unroll=False)
        return ()

    lax.fori_loop(0, TPW, task_body, (), unroll=False)

    for k in range(NBUF):
        pltpu.make_async_copy(
            out_hbm.at[0, 0, pl.ds(0, AC)], bufs[k], sems[k]).wait()


def kernel(seq1M, seq2M, patches, geo):
    seq1p = seq1M.reshape(B, SR // 2, 2 * D)
    seq2p = seq2M.reshape(B, SL // 2, 2 * D)
    geo8 = geo.reshape(B, P, PS * PS // 128, 128)
    r = patches[:, :, 0].reshape(TASKS).astype(jnp.int32)
    cc = patches[:, :, 1].reshape(TASKS).astype(jnp.int32)
    q8r = jnp.minimum((r // 16) * 8, SR // 2 - SPAIRS)
    q8c = jnp.minimum((cc // 16) * 8, SL // 2 - SPAIRS)
    desc = jnp.stack([q8r, r - 2 * q8r, q8c, cc - 2 * q8c], axis=1)
    desc = jnp.pad(desc, ((0, 0), (0, 128 - 4)))

    run = pl.kernel(
        _sc_body,
        out_type=jax.ShapeDtypeStruct((B, P, PS, PS, CH), jnp.float32),
        mesh=plsc.VectorSubcoreMesh(core_axis_name="c", subcore_axis_name="s"),
        compiler_params=pltpu.CompilerParams(needs_layout_passes=False),
        scratch_types=[
            pltpu.VMEM((128,), jnp.int32),
            pltpu.VMEM((SPAIRS, 2 * D), jnp.float32),
            pltpu.VMEM((SPAIRS, 2 * D), jnp.float32),
            pltpu.VMEM((PS * PS // 128, 128), jnp.float32),
            pltpu.VMEM((AC, PS, CH), jnp.float32),
            pltpu.VMEM((AC, PS, CH), jnp.float32),
            pltpu.SemaphoreType.DMA,
            pltpu.SemaphoreType.DMA,
            pltpu.SemaphoreType.DMA,
        ],
    )
    return run(seq1p, seq2p, desc, geo8)
